# Initial kernel scaffold; baseline (speedup 1.0000x reference)
import functools

import jax
import jax.numpy as jnp
from jax import lax
from jax.experimental import pallas as pl
from jax.experimental.pallas import tpu as pltpu
from jax.experimental.pallas import tpu_sc as plsc

NC = 2
NS = 16
NW = NC * NS

IDX_MINOR = 128
CHUNK = 1024
RPG = CHUNK // IDX_MINOR


@functools.partial(jax.jit, static_argnums=(2, 3))
def _embed_lookup(idx2d, table, b_total, hidden):
    b_per_w = b_total // NW
    n_chunks = b_per_w // CHUNK
    mesh = plsc.VectorSubcoreMesh(core_axis_name="c", subcore_axis_name="s")

    @functools.partial(
        pl.kernel,
        out_type=jax.ShapeDtypeStruct((b_total, hidden), jnp.float32),
        mesh=mesh,
        scratch_types=[
            pltpu.VMEM((RPG, IDX_MINOR), jnp.int32),
            pltpu.VMEM((CHUNK, hidden), jnp.float32),
            pltpu.SemaphoreType.DMA,
        ],
    )
    def body(idx_hbm, table_hbm, out_hbm, idx_v, rows_v, sem):
        wid = lax.axis_index("s") * NC + lax.axis_index("c")
        row_base = wid * (b_per_w // IDX_MINOR)

        def chunk_body(g, carry):
            pltpu.sync_copy(idx_hbm.at[pl.ds(row_base + g * RPG, RPG)], idx_v)
            copies = [
                pltpu.async_copy(
                    table_hbm.at[idx_v.at[j]],
                    rows_v.at[pl.ds(j * IDX_MINOR, IDX_MINOR)],
                    sem,
                )
                for j in range(RPG)
            ]
            for c in copies:
                c.wait()
            pltpu.sync_copy(
                rows_v,
                out_hbm.at[pl.ds(wid * b_per_w + g * CHUNK, CHUNK)],
            )
            return carry

        lax.fori_loop(0, n_chunks, chunk_body, 0)

    return body(idx2d, table)


def kernel(input_ids, W_embed):
    batch, hist = input_ids.shape
    hidden = W_embed.shape[1]
    b_total = batch * hist
    idx2d = input_ids.reshape(b_total // IDX_MINOR, IDX_MINOR)
    out = _embed_lookup(idx2d, W_embed, b_total, hidden)
    return out.reshape(batch, hist, hidden)


# speedup vs baseline: 4.8070x; 4.8070x over previous
import functools

import jax
import jax.numpy as jnp
from jax import lax
from jax.experimental import pallas as pl
from jax.experimental.pallas import tpu as pltpu
from jax.experimental.pallas import tpu_sc as plsc

NC = 2
NS = 16
NW = NC * NS

IDX_MINOR = 128
CHUNK = 1024
RPG = CHUNK // IDX_MINOR


@functools.partial(jax.jit, static_argnums=(2, 3))
def _embed_lookup(idx2d, table, b_total, hidden):
    b_per_w = b_total // NW
    n_chunks = b_per_w // CHUNK
    mesh = plsc.VectorSubcoreMesh(core_axis_name="c", subcore_axis_name="s")

    @functools.partial(
        pl.kernel,
        out_type=jax.ShapeDtypeStruct((b_total, hidden), jnp.float32),
        mesh=mesh,
        scratch_types=[
            pltpu.VMEM((RPG, IDX_MINOR), jnp.int32),
            pltpu.VMEM((CHUNK, hidden), jnp.float32),
            pltpu.SemaphoreType.DMA,
        ],
        compiler_params=pltpu.CompilerParams(use_tc_tiling_on_sc=False),
    )
    def body(idx_hbm, table_hbm, out_hbm, idx_v, rows_v, sem):
        wid = lax.axis_index("s") * NC + lax.axis_index("c")
        row_base = wid * (b_per_w // IDX_MINOR)

        def chunk_body(g, carry):
            pltpu.sync_copy(idx_hbm.at[pl.ds(row_base + g * RPG, RPG)], idx_v)
            copies = [
                pltpu.async_copy(
                    table_hbm.at[idx_v.at[j]],
                    rows_v.at[pl.ds(j * IDX_MINOR, IDX_MINOR)],
                    sem,
                )
                for j in range(RPG)
            ]
            for c in copies:
                c.wait()
            pltpu.sync_copy(
                rows_v,
                out_hbm.at[pl.ds(wid * b_per_w + g * CHUNK, CHUNK)],
            )
            return carry

        lax.fori_loop(0, n_chunks, chunk_body, 0)

    return body(idx2d, table)


def kernel(input_ids, W_embed):
    batch, hist = input_ids.shape
    hidden = W_embed.shape[1]
    b_total = batch * hist
    idx2d = input_ids.reshape(b_total // IDX_MINOR, IDX_MINOR)
    out = _embed_lookup(idx2d, W_embed, b_total, hidden)
    return out.reshape(batch, hist, hidden)


# speedup vs baseline: 5.0329x; 1.0470x over previous
import functools

import jax
import jax.numpy as jnp
from jax import lax
from jax.experimental import pallas as pl
from jax.experimental.pallas import tpu as pltpu
from jax.experimental.pallas import tpu_sc as plsc

NC = 2
NS = 16
NW = NC * NS

IDX_MINOR = 128
CHUNK = 1024
RPG = CHUNK // IDX_MINOR
NBUF = 2


@functools.partial(jax.jit, static_argnums=(2, 3))
def _embed_lookup(idx2d, table, b_total, hidden):
    b_per_w = b_total // NW
    n_chunks = b_per_w // CHUNK
    assert n_chunks % NBUF == 0
    mesh = plsc.VectorSubcoreMesh(core_axis_name="c", subcore_axis_name="s")

    @functools.partial(
        pl.kernel,
        out_type=jax.ShapeDtypeStruct((b_total, hidden), jnp.float32),
        mesh=mesh,
        scratch_types=[
            pltpu.VMEM((NBUF, RPG, IDX_MINOR), jnp.int32),
            pltpu.VMEM((NBUF, CHUNK, hidden), jnp.float32),
            pltpu.SemaphoreType.DMA((NBUF,)),
            pltpu.SemaphoreType.DMA((NBUF,)),
            pltpu.SemaphoreType.DMA((NBUF,)),
        ],
        compiler_params=pltpu.CompilerParams(use_tc_tiling_on_sc=False),
    )
    def body(idx_hbm, table_hbm, out_hbm, idx_v, rows_v, idx_sem, gat_sem, out_sem):
        wid = lax.axis_index("s") * NC + lax.axis_index("c")
        row_base = wid * (b_per_w // IDX_MINOR)
        out_base = wid * b_per_w

        def start_idx(g, b):
            pltpu.async_copy(
                idx_hbm.at[pl.ds(row_base + g * RPG, RPG)],
                idx_v.at[b],
                idx_sem.at[b],
            )

        def wait_idx(b):
            pltpu.make_async_copy(
                idx_hbm.at[pl.ds(row_base, RPG)], idx_v.at[b], idx_sem.at[b]
            ).wait()

        def wait_out(b):
            pltpu.make_async_copy(
                rows_v.at[b], out_hbm.at[pl.ds(out_base, CHUNK)], out_sem.at[b]
            ).wait()

        for b in range(NBUF):
            start_idx(b, b)

        def outer(go, carry):
            for b in range(NBUF):
                g = go * NBUF + b
                wait_idx(b)
                @pl.when(g >= NBUF)
                def _():
                    wait_out(b)
                copies = [
                    pltpu.async_copy(
                        table_hbm.at[idx_v.at[b].at[j]],
                        rows_v.at[b].at[pl.ds(j * IDX_MINOR, IDX_MINOR)],
                        gat_sem.at[b],
                    )
                    for j in range(RPG)
                ]
                for c in copies:
                    c.wait()
                @pl.when(g + NBUF < n_chunks)
                def _():
                    start_idx(g + NBUF, b)
                pltpu.async_copy(
                    rows_v.at[b],
                    out_hbm.at[pl.ds(out_base + g * CHUNK, CHUNK)],
                    out_sem.at[b],
                )
            return carry

        lax.fori_loop(0, n_chunks // NBUF, outer, 0)

        for b in range(NBUF):
            wait_out(b)

    return body(idx2d, table)


def kernel(input_ids, W_embed):
    batch, hist = input_ids.shape
    hidden = W_embed.shape[1]
    b_total = batch * hist
    idx2d = input_ids.reshape(b_total // IDX_MINOR, IDX_MINOR)
    out = _embed_lookup(idx2d, W_embed, b_total, hidden)
    return out.reshape(batch, hist, hidden)
